# baseline (speedup 1.0000x reference)
import jax
import jax.numpy as jnp
from jax import lax
from jax.experimental import pallas as pl
from jax.experimental.pallas import tpu as pltpu
from jax.experimental.pallas import tpu_sc as plsc

_ROWS = 32768
_COLS = 2048

_SC_ROWS = 4096
_TC_ROWS = _ROWS - _SC_ROWS
_TC_BLOCK = 1024
_TC_GRID = _TC_ROWS // _TC_BLOCK

_NC = 2
_NS = 16
_NW = _NC * _NS
_ROWS_PER_W = _SC_ROWS // _NW
_STRIPE = 8
_SC_ITERS = _ROWS_PER_W // _STRIPE
_UNROLL = 4


def _tc_body(x_ref, y_ref, out_ref, acc_ref):
    i = pl.program_id(0)

    @pl.when(i == 0)
    def _init():
        acc_ref[0] = 0.0
        acc_ref[1] = 0.0

    x = x_ref[...]
    y = y_ref[...]
    d = x - y
    acc_ref[0] += jnp.sum(d * d)
    acc_ref[1] += jnp.sum(y * y)

    @pl.when(i == _TC_GRID - 1)
    def _finish():
        out_ref[0] = acc_ref[0]
        out_ref[1] = acc_ref[1]


def _sc_body(x_hbm, y_hbm, out_hbm, xv0, yv0, xv1, yv1, ov, sem0, sem1):
    c = lax.axis_index("c")
    s = lax.axis_index("s")
    wid = s * _NC + c
    base_row = _TC_ROWS + wid * _ROWS_PER_W

    def row_of(i):
        return jnp.minimum(base_row + i * _STRIPE, _ROWS - _STRIPE)

    def start(i, xv, yv, sem):
        r = row_of(i)
        pltpu.async_copy(x_hbm.at[pl.ds(r, _STRIPE)], xv, sem)
        pltpu.async_copy(y_hbm.at[pl.ds(r, _STRIPE)], yv, sem)

    def drain(xv, yv, sem):
        pltpu.make_async_copy(x_hbm.at[pl.ds(0, _STRIPE)], xv, sem).wait()
        pltpu.make_async_copy(y_hbm.at[pl.ds(0, _STRIPE)], yv, sem).wait()

    def process(xv, yv, acc_d, acc_y):
        def inner(k, accs2):
            ad, ay = accs2
            for j in range(_UNROLL):
                col = (k * _UNROLL + j) * 16
                for r in range(_STRIPE):
                    xx = xv[r, pl.ds(col, 16)]
                    yy = yv[r, pl.ds(col, 16)]
                    d = xx - yy
                    ad = ad + d * d
                    ay = ay + yy * yy
            return (ad, ay)

        return lax.fori_loop(0, _COLS // (16 * _UNROLL), inner,
                             (acc_d, acc_y))

    start(0, xv0, yv0, sem0)

    def body(m, accs):
        acc_d, acc_y = accs
        i0 = 2 * m
        drain(xv0, yv0, sem0)
        start(i0 + 1, xv1, yv1, sem1)
        acc_d, acc_y = process(xv0, yv0, acc_d, acc_y)
        drain(xv1, yv1, sem1)
        start(i0 + 2, xv0, yv0, sem0)
        acc_d, acc_y = process(xv1, yv1, acc_d, acc_y)
        return (acc_d, acc_y)

    zero = jnp.zeros((16,), jnp.float32)
    acc_d, acc_y = lax.fori_loop(0, _SC_ITERS // 2, body, (zero, zero))
    drain(xv0, yv0, sem0)
    ov[pl.ds(0, 16)] = acc_d
    ov[pl.ds(16, 16)] = acc_y
    pltpu.sync_copy(ov, out_hbm.at[wid])


def kernel(x, y):
    x2 = x.reshape(_ROWS, _COLS)
    y2 = y.reshape(_ROWS, _COLS)

    mesh = plsc.VectorSubcoreMesh(core_axis_name="c", subcore_axis_name="s")
    sc_partials = pl.kernel(
        _sc_body,
        mesh=mesh,
        out_type=jax.ShapeDtypeStruct((_NW, 32), jnp.float32),
        scratch_types=[
            pltpu.VMEM((_STRIPE, _COLS), jnp.float32),
            pltpu.VMEM((_STRIPE, _COLS), jnp.float32),
            pltpu.VMEM((_STRIPE, _COLS), jnp.float32),
            pltpu.VMEM((_STRIPE, _COLS), jnp.float32),
            pltpu.VMEM((32,), jnp.float32),
            pltpu.SemaphoreType.DMA,
            pltpu.SemaphoreType.DMA,
        ],
        compiler_params=pltpu.CompilerParams(use_tc_tiling_on_sc=True),
    )(x2, y2)

    tc_sums = pl.pallas_call(
        _tc_body,
        grid=(_TC_GRID,),
        in_specs=[
            pl.BlockSpec((_TC_BLOCK, _COLS), lambda i: (i, 0)),
            pl.BlockSpec((_TC_BLOCK, _COLS), lambda i: (i, 0)),
        ],
        out_specs=pl.BlockSpec(memory_space=pltpu.SMEM),
        out_shape=jax.ShapeDtypeStruct((2,), jnp.float32),
        scratch_shapes=[pltpu.SMEM((2,), jnp.float32)],
    )(x2, y2)

    sum_d = tc_sums[0] + jnp.sum(sc_partials[:, :16])
    sum_y = tc_sums[1] + jnp.sum(sc_partials[:, 16:])
    return sum_d / (2.0 * sum_y)


# speedup vs baseline: 1.1533x; 1.1533x over previous
import jax
import jax.numpy as jnp
from jax.experimental import pallas as pl
from jax.experimental.pallas import tpu as pltpu

_ROWS = 32768
_COLS = 2048
_BLOCK_ROWS = 1024
_GRID = _ROWS // _BLOCK_ROWS


def _loss_kernel(x_ref, y_ref, out_ref, acc_ref):
    i = pl.program_id(0)

    @pl.when(i == 0)
    def _init():
        acc_ref[0] = 0.0
        acc_ref[1] = 0.0

    x = x_ref[...]
    y = y_ref[...]
    d = x - y
    acc_ref[0] += jnp.sum(d * d)
    acc_ref[1] += jnp.sum(y * y)

    @pl.when(i == _GRID - 1)
    def _finish():
        out_ref[0] = acc_ref[0] / (2.0 * acc_ref[1])


def kernel(x, y):
    x2 = x.reshape(_ROWS, _COLS)
    y2 = y.reshape(_ROWS, _COLS)
    out = pl.pallas_call(
        _loss_kernel,
        grid=(_GRID,),
        in_specs=[
            pl.BlockSpec((_BLOCK_ROWS, _COLS), lambda i: (i, 0)),
            pl.BlockSpec((_BLOCK_ROWS, _COLS), lambda i: (i, 0)),
        ],
        out_specs=pl.BlockSpec(memory_space=pltpu.SMEM),
        out_shape=jax.ShapeDtypeStruct((1,), jnp.float32),
        scratch_shapes=[pltpu.SMEM((2,), jnp.float32)],
    )(x2, y2)
    return out[0]


# speedup vs baseline: 1.1588x; 1.0048x over previous
import jax
import jax.numpy as jnp
from jax import lax
from jax.experimental import pallas as pl
from jax.experimental.pallas import tpu as pltpu

_ROWS = 32768
_COLS = 2048
_CH = 512
_NCHUNK = _ROWS // _CH
_DEPTH = 4


def _loss_kernel(x_hbm, y_hbm, out_ref, xb, yb, sx, sy):
    def start(i):
        slot = lax.rem(i, _DEPTH)
        pltpu.make_async_copy(
            x_hbm.at[pl.ds(i * _CH, _CH)], xb.at[slot], sx.at[slot]).start()
        pltpu.make_async_copy(
            y_hbm.at[pl.ds(i * _CH, _CH)], yb.at[slot], sy.at[slot]).start()

    for i in range(_DEPTH):
        start(i)

    def step(i, accs):
        a_d, a_y = accs
        slot = lax.rem(i, _DEPTH)
        pltpu.make_async_copy(
            x_hbm.at[pl.ds(i * _CH, _CH)], xb.at[slot], sx.at[slot]).wait()
        pltpu.make_async_copy(
            y_hbm.at[pl.ds(i * _CH, _CH)], yb.at[slot], sy.at[slot]).wait()
        x = xb[slot]
        y = yb[slot]
        d = x - y
        a_d = a_d + jnp.sum(d * d)
        a_y = a_y + jnp.sum(y * y)

        @pl.when(i + _DEPTH < _NCHUNK)
        def _():
            start(i + _DEPTH)

        return (a_d, a_y)

    a_d, a_y = lax.fori_loop(0, _NCHUNK, step, (0.0, 0.0))
    out_ref[0] = a_d / (2.0 * a_y)


def kernel(x, y):
    x2 = x.reshape(_ROWS, _COLS)
    y2 = y.reshape(_ROWS, _COLS)
    out = pl.pallas_call(
        _loss_kernel,
        in_specs=[
            pl.BlockSpec(memory_space=pl.ANY),
            pl.BlockSpec(memory_space=pl.ANY),
        ],
        out_specs=pl.BlockSpec(memory_space=pltpu.SMEM),
        out_shape=jax.ShapeDtypeStruct((1,), jnp.float32),
        scratch_shapes=[
            pltpu.VMEM((_DEPTH, _CH, _COLS), jnp.float32),
            pltpu.VMEM((_DEPTH, _CH, _COLS), jnp.float32),
            pltpu.SemaphoreType.DMA((_DEPTH,)),
            pltpu.SemaphoreType.DMA((_DEPTH,)),
        ],
    )(x2, y2)
    return out[0]


# speedup vs baseline: 1.1593x; 1.0004x over previous
import jax
import jax.numpy as jnp
from jax import lax
from jax.experimental import pallas as pl
from jax.experimental.pallas import tpu as pltpu

_ROWS = 32768
_COLS = 2048
_CH = 256
_NCHUNK = _ROWS // _CH
_DEPTH = 8


def _loss_kernel(x_hbm, y_hbm, out_ref, xb, yb, sx, sy):
    def start(i):
        slot = lax.rem(i, _DEPTH)
        pltpu.make_async_copy(
            x_hbm.at[pl.ds(i * _CH, _CH)], xb.at[slot], sx.at[slot]).start()
        pltpu.make_async_copy(
            y_hbm.at[pl.ds(i * _CH, _CH)], yb.at[slot], sy.at[slot]).start()

    for i in range(_DEPTH):
        start(i)

    def step(i, accs):
        a_d, a_y = accs
        slot = lax.rem(i, _DEPTH)
        pltpu.make_async_copy(
            x_hbm.at[pl.ds(i * _CH, _CH)], xb.at[slot], sx.at[slot]).wait()
        pltpu.make_async_copy(
            y_hbm.at[pl.ds(i * _CH, _CH)], yb.at[slot], sy.at[slot]).wait()
        x = xb[slot]
        y = yb[slot]
        d = x - y
        a_d = a_d + jnp.sum(d * d)
        a_y = a_y + jnp.sum(y * y)

        @pl.when(i + _DEPTH < _NCHUNK)
        def _():
            start(i + _DEPTH)

        return (a_d, a_y)

    a_d, a_y = lax.fori_loop(0, _NCHUNK, step, (0.0, 0.0))
    out_ref[0] = a_d / (2.0 * a_y)


def kernel(x, y):
    x2 = x.reshape(_ROWS, _COLS)
    y2 = y.reshape(_ROWS, _COLS)
    out = pl.pallas_call(
        _loss_kernel,
        in_specs=[
            pl.BlockSpec(memory_space=pl.ANY),
            pl.BlockSpec(memory_space=pl.ANY),
        ],
        out_specs=pl.BlockSpec(memory_space=pltpu.SMEM),
        out_shape=jax.ShapeDtypeStruct((1,), jnp.float32),
        scratch_shapes=[
            pltpu.VMEM((_DEPTH, _CH, _COLS), jnp.float32),
            pltpu.VMEM((_DEPTH, _CH, _COLS), jnp.float32),
            pltpu.SemaphoreType.DMA((_DEPTH,)),
            pltpu.SemaphoreType.DMA((_DEPTH,)),
        ],
    )(x2, y2)
    return out[0]
